# baseline (speedup 1.0000x reference)
import functools

import jax
import jax.numpy as jnp
from jax import lax
from jax.experimental import pallas as pl
from jax.experimental.pallas import tpu as pltpu
from jax.experimental.pallas import tpu_sc as plsc

_B = 16384
_D = 32
_NC = 2
_NS = 16
_NW = _NC * _NS
_PW = _B // _NW
_CH = 128
_NCH = _PW // _CH


@functools.partial(
    pl.kernel,
    mesh=plsc.VectorSubcoreMesh(core_axis_name="c", subcore_axis_name="s"),
    out_type=jax.ShapeDtypeStruct((_B, 2, _D), jnp.float32),
    compiler_params=pltpu.CompilerParams(use_tc_tiling_on_sc=False),
    scratch_types=[
        pltpu.VMEM((_NCH, _CH), jnp.int32),
        pltpu.VMEM((_NCH, _CH), jnp.int32),
        pltpu.VMEM((_PW, _D), jnp.float32),
        pltpu.VMEM((_PW, _D), jnp.float32),
        pltpu.SemaphoreType.DMA,
    ],
)
def _edge_gather(src_hbm, dst_hbm, table_hbm, out_hbm,
                 idx_s, idx_d, rows_s, rows_d, sem):
    wid = lax.axis_index("s") * _NC + lax.axis_index("c")
    pltpu.sync_copy(src_hbm.at[wid], idx_s)
    pltpu.sync_copy(dst_hbm.at[wid], idx_d)

    copies = []
    for j in range(_NCH):
        copies.append(pltpu.async_copy(
            table_hbm.at[idx_s.at[j]], rows_s.at[pl.ds(j * _CH, _CH)], sem))
        copies.append(pltpu.async_copy(
            table_hbm.at[idx_d.at[j]], rows_d.at[pl.ds(j * _CH, _CH)], sem))
    for c in copies:
        c.wait()

    base = wid * _PW
    pltpu.sync_copy(rows_s, out_hbm.at[pl.ds(base, _PW), 0])
    pltpu.sync_copy(rows_d, out_hbm.at[pl.ds(base, _PW), 1])


def kernel(source_node_input, destination_node_input, table):
    src = source_node_input.reshape(_NW, _NCH, _CH)
    dst = destination_node_input.reshape(_NW, _NCH, _CH)
    rows = _edge_gather(src, dst, table)
    return rows.reshape(_B, 2 * _D)


# speedup vs baseline: 2.1097x; 2.1097x over previous
import functools

import jax
import jax.numpy as jnp
from jax import lax
from jax.experimental import pallas as pl
from jax.experimental.pallas import tpu as pltpu
from jax.experimental.pallas import tpu_sc as plsc

_B = 16384
_D = 32
_NC = 2
_NS = 16
_NW = _NC * _NS
_PW = _B // _NW
_K = 16
_NB = _PW // _K


@functools.partial(
    pl.kernel,
    mesh=plsc.VectorSubcoreMesh(core_axis_name="c", subcore_axis_name="s"),
    out_type=jax.ShapeDtypeStruct((2 * _D, _B), jnp.float32),
    compiler_params=pltpu.CompilerParams(use_tc_tiling_on_sc=True, needs_layout_passes=False),
    scratch_types=[
        pltpu.VMEM((_PW,), jnp.int32),
        pltpu.VMEM((_PW,), jnp.int32),
        pltpu.VMEM((1, 8, 128), jnp.int32),
        pltpu.VMEM((_K,), jnp.int32),
        pltpu.VMEM((_K, _D, 128), jnp.float32),
        pltpu.VMEM((_D, _PW), jnp.float32),
        pltpu.VMEM((_D, _PW), jnp.float32),
        pltpu.SemaphoreType.DMA,
    ],
)
def _edge_gather(src_hbm, dst_hbm, table_t_hbm, out_hbm,
                 idx_s, idx_d, lanes, ebase, win, cols_s, cols_d, sem):
    wid = lax.axis_index("s") * _NC + lax.axis_index("c")
    pltpu.sync_copy(src_hbm.at[wid], idx_s)
    pltpu.sync_copy(dst_hbm.at[wid], idx_d)

    d_lo = lax.iota(jnp.int32, 16)
    d_hi = d_lo + 16

    def stream(idx_ref, cols_ref):
        ebase[...] = jnp.zeros((_K,), jnp.int32)

        def batch(b, carry):
            iv = idx_ref[pl.ds(b * _K, _K)]
            for j in range(_K):
                t0 = pl.multiple_of((iv[j] >> 7) * 128, 128)
                pltpu.async_copy(
                    table_t_hbm.at[:, pl.ds(t0, 128)], win.at[j], sem)
            for j in range(_K):
                pltpu.make_async_copy(
                    table_t_hbm.at[:, pl.ds(0, 128)], win.at[j], sem).wait()
            for j in range(_K):
                j16 = jnp.full((16,), j, jnp.int32)
                l16 = jnp.full((16,), iv[j] & 127, jnp.int32)
                e16 = jnp.full((16,), b * _K + j, jnp.int32)
                v_lo = plsc.load_gather(win, [j16, d_lo, l16])
                v_hi = plsc.load_gather(win, [j16, d_hi, l16])
                plsc.store_scatter(cols_ref, [d_lo, e16], v_lo)
                plsc.store_scatter(cols_ref, [d_hi, e16], v_hi)
            return carry

        lax.fori_loop(0, _NB, batch, 0)

    stream(idx_s, cols_s)
    stream(idx_d, cols_d)

    base = wid * _PW
    pltpu.sync_copy(cols_s, out_hbm.at[pl.ds(0, _D), pl.ds(base, _PW)])
    pltpu.sync_copy(cols_d, out_hbm.at[pl.ds(_D, _D), pl.ds(base, _PW)])


def kernel(source_node_input, destination_node_input, table):
    src = source_node_input.reshape(_NW, _PW)
    dst = destination_node_input.reshape(_NW, _PW)
    out_t = _edge_gather(src, dst, table.T)
    return out_t.T


# speedup vs baseline: 2.4147x; 1.1446x over previous
import functools

import jax
import jax.numpy as jnp
from jax import lax
from jax.experimental import pallas as pl
from jax.experimental.pallas import tpu as pltpu
from jax.experimental.pallas import tpu_sc as plsc

_B = 16384
_D = 32
_NC = 2
_NS = 16
_NW = _NC * _NS
_PW = _B // _NW
_K = 16
_NB = _PW // _K


@functools.partial(
    pl.kernel,
    mesh=plsc.VectorSubcoreMesh(core_axis_name="c", subcore_axis_name="s"),
    out_type=jax.ShapeDtypeStruct((2 * _D, _B), jnp.float32),
    compiler_params=pltpu.CompilerParams(
        use_tc_tiling_on_sc=True, needs_layout_passes=False),
    scratch_types=[
        pltpu.VMEM((_PW,), jnp.int32),
        pltpu.VMEM((_PW,), jnp.int32),
        pltpu.VMEM((_K, _D, 128), jnp.float32),
        pltpu.VMEM((_D, _PW), jnp.float32),
        pltpu.VMEM((_D, _PW), jnp.float32),
    ] + [pltpu.SemaphoreType.DMA] * _K,
)
def _edge_gather(src_hbm, dst_hbm, table_t_hbm, out_hbm,
                 idx_s, idx_d, win, cols_s, cols_d, *sems):
    wid = lax.axis_index("s") * _NC + lax.axis_index("c")
    pltpu.sync_copy(src_hbm.at[wid], idx_s)
    pltpu.sync_copy(dst_hbm.at[wid], idx_d)

    d_lo = lax.iota(jnp.int32, 16)
    d_hi = d_lo + 16

    def enqueue(i, j):
        t0 = pl.multiple_of((i >> 7) * 128, 128)
        pltpu.async_copy(
            table_t_hbm.at[:, pl.ds(t0, 128)], win.at[j], sems[j])

    def extract(cols_ref, i, e, j):
        pltpu.make_async_copy(
            table_t_hbm.at[:, pl.ds(0, 128)], win.at[j], sems[j]).wait()
        j16 = jnp.full((16,), j, jnp.int32)
        l16 = jnp.full((16,), i & 127, jnp.int32)
        e16 = jnp.full((16,), e, jnp.int32)
        v_lo = plsc.load_gather(win, [j16, d_lo, l16])
        v_hi = plsc.load_gather(win, [j16, d_hi, l16])
        plsc.store_scatter(cols_ref, [d_lo, e16], v_lo)
        plsc.store_scatter(cols_ref, [d_hi, e16], v_hi)

    def stream(idx_ref, cols_ref):
        iv0 = idx_ref[pl.ds(0, _K)]
        for j in range(_K):
            enqueue(iv0[j], j)

        def round_(b, carry):
            iv_prev = idx_ref[pl.ds((b - 1) * _K, _K)]
            iv_cur = idx_ref[pl.ds(b * _K, _K)]
            for j in range(_K):
                extract(cols_ref, iv_prev[j], (b - 1) * _K + j, j)
                enqueue(iv_cur[j], j)
            return carry

        lax.fori_loop(1, _NB, round_, 0)
        iv_last = idx_ref[pl.ds((_NB - 1) * _K, _K)]
        for j in range(_K):
            extract(cols_ref, iv_last[j], (_NB - 1) * _K + j, j)

    stream(idx_s, cols_s)
    stream(idx_d, cols_d)

    base = wid * _PW
    pltpu.sync_copy(cols_s, out_hbm.at[pl.ds(0, _D), pl.ds(base, _PW)])
    pltpu.sync_copy(cols_d, out_hbm.at[pl.ds(_D, _D), pl.ds(base, _PW)])


def kernel(source_node_input, destination_node_input, table):
    src = source_node_input.reshape(_NW, _PW)
    dst = destination_node_input.reshape(_NW, _PW)
    out_t = _edge_gather(src, dst, table.T)
    return out_t.T
